# baseline (speedup 1.0000x reference)
import functools

import jax
import jax.numpy as jnp
from jax import lax
from jax.experimental import pallas as pl
from jax.experimental.pallas import tpu as pltpu
from jax.experimental.pallas import tpu_sc as plsc

S = 2048
M = 768
E = 8
MID = 3072
OUT = 768
C = 512
EC = E * C
EP = 16
LANES = 128
NW = 32
L = 16
ROWS_DISP = EC // NW
ROWS_COMB = S // NW
BM = 1024
KM = MID // BM
RB = 256
NR = C // RB
SENT_TOK = S
SENT_SLOT = EC


def _gating_body(x_ref, wg_ref, dst_ref, gate_ref, cnt_ref, laux_ref):
    x = x_ref[...]
    wg = wg_ref[...]
    logits = jnp.dot(x, wg, preferred_element_type=jnp.float32)
    col = lax.broadcasted_iota(jnp.int32, (S, LANES), 1)
    valid = col < E
    logits = jnp.where(valid, logits, -1e30)
    mx = jnp.max(logits, axis=1, keepdims=True)
    p = jnp.exp(logits - mx)
    probs = p / jnp.sum(p, axis=1, keepdims=True)
    pm = jnp.max(probs, axis=1, keepdims=True)
    is_max = probs == pm
    eidx = jnp.min(jnp.where(is_max, col, LANES), axis=1, keepdims=True)
    onehot = jnp.where((col == eidx) & valid, 1.0, 0.0)
    c = onehot
    k = 1
    while k < S:
        shifted = jnp.concatenate(
            [jnp.zeros((k, LANES), jnp.float32), c[: S - k, :]], axis=0)
        c = c + shifted
        k *= 2
    pos = jnp.sum(c * onehot, axis=1, keepdims=True).astype(jnp.int32) - 1
    kept = pos < C
    dst = jnp.where(kept, eidx * C + pos, SENT_SLOT)
    dst_ref[...] = dst.astype(jnp.int32)
    gate_ref[...] = jnp.sum(probs * onehot, axis=1, keepdims=True)
    counts = jnp.sum(onehot, axis=0, keepdims=True)
    cnt_ref[...] = counts.astype(jnp.int32)
    me = jnp.sum(probs, axis=0, keepdims=True) / S
    laux_ref[...] = jnp.sum(me * (counts / S), axis=1, keepdims=True) * E


def _gating(x, wg_pad):
    return pl.pallas_call(
        _gating_body,
        out_shape=(
            jax.ShapeDtypeStruct((S, 1), jnp.int32),
            jax.ShapeDtypeStruct((S, 1), jnp.float32),
            jax.ShapeDtypeStruct((1, LANES), jnp.int32),
            jax.ShapeDtypeStruct((1, 1), jnp.float32),
        ),
    )(x, wg_pad)


def _dispatch_body(dst_hbm, gate_hbm, feat_hbm, disp_hbm, scale_hbm,
                   dst_v, gate_v, slot_v, scale_v, rows_v, sem):
    wid = lax.axis_index("s") * 2 + lax.axis_index("c")
    base = wid * ROWS_DISP
    pltpu.sync_copy(dst_hbm, dst_v)
    pltpu.sync_copy(gate_hbm, gate_v)
    for j in range(ROWS_DISP // L):
        dflt = (lax.iota(jnp.int32, L) + (base + j * L)) & (S - 1)
        slot_v[pl.ds(j * L, L)] = dflt

    def build(i, _):
        idx = dst_v[pl.ds(i * L, L)] - base
        m = (idx >= 0) & (idx < ROWS_DISP)
        tok = lax.iota(jnp.int32, L) + i * L
        plsc.store_scatter(slot_v, [idx], tok, mask=m)
        return 0
    lax.fori_loop(0, S // L, build, 0)

    for j in range(ROWS_DISP // L):
        sidx = slot_v[pl.ds(j * L, L)]
        scale_v[pl.ds(j * L, L)] = plsc.load_gather(gate_v, [sidx])
    pltpu.sync_copy(scale_v, scale_hbm.at[pl.ds(base, ROWS_DISP)])

    pltpu.async_copy(feat_hbm.at[slot_v], rows_v, sem).wait()
    pltpu.sync_copy(rows_v, disp_hbm.at[pl.ds(base, ROWS_DISP)])


def _dispatch(dst, gate_pad, feat_pad):
    mesh = plsc.VectorSubcoreMesh(core_axis_name="c", subcore_axis_name="s")
    return pl.kernel(
        _dispatch_body,
        out_type=(
            jax.ShapeDtypeStruct((EC, M), jnp.float32),
            jax.ShapeDtypeStruct((EC,), jnp.float32),
        ),
        mesh=mesh,
        compiler_params=pltpu.CompilerParams(needs_layout_passes=False),
        scratch_types=[
            pltpu.VMEM((S,), jnp.int32),
            pltpu.VMEM((S,), jnp.float32),
            pltpu.VMEM((ROWS_DISP,), jnp.int32),
            pltpu.VMEM((ROWS_DISP,), jnp.float32),
            pltpu.VMEM((ROWS_DISP, M), jnp.float32),
            pltpu.SemaphoreType.DMA,
        ],
    )(dst, gate_pad, feat_pad)


def _ffn_body(cnt_ref, x_ref, w1_ref, b1_ref, w2_ref, b2_ref, sc_ref,
              o_ref, acc_ref):
    e = pl.program_id(0)
    k = pl.program_id(1)
    r = pl.program_id(2)
    rs = r * RB
    n = cnt_ref[e]

    @pl.when(n > rs)
    def _():
        @pl.when(k == 0)
        def _():
            acc_ref[pl.ds(rs, RB), :] = jnp.zeros((RB, OUT), jnp.float32)

        x = x_ref[pl.ds(rs, RB), :].astype(jnp.bfloat16)
        h = jnp.dot(x, w1_ref[0].astype(jnp.bfloat16),
                    preferred_element_type=jnp.float32)
        h = jnp.maximum(h + b1_ref[0, 0, 0][None, :], 0.0)
        acc_ref[pl.ds(rs, RB), :] += jnp.dot(
            h.astype(jnp.bfloat16), w2_ref[0].astype(jnp.bfloat16),
            preferred_element_type=jnp.float32)

    @pl.when(k == KM - 1)
    def _():
        o_ref[pl.ds(rs, RB), :] = (
            (acc_ref[pl.ds(rs, RB), :] + b2_ref[0, 0][None, :])
            * sc_ref[pl.ds(rs, RB), :])


def _ffn(cnt, disp, w1, b1, w2, b2, scale):
    ce = lambda e: jnp.minimum(e, E - 1)
    grid_spec = pltpu.PrefetchScalarGridSpec(
        num_scalar_prefetch=1,
        grid=(E + 1, KM, NR),
        in_specs=[
            pl.BlockSpec((C, M), lambda e, k, r, cnt: (ce(e), 0)),
            pl.BlockSpec((1, M, BM), lambda e, k, r, cnt: (ce(e), 0, k)),
            pl.BlockSpec((1, 1, 1, BM), lambda e, k, r, cnt: (ce(e), k, 0, 0)),
            pl.BlockSpec((1, BM, OUT), lambda e, k, r, cnt: (ce(e), k, 0)),
            pl.BlockSpec((1, 1, OUT), lambda e, k, r, cnt: (ce(e), 0, 0)),
            pl.BlockSpec((C, 1), lambda e, k, r, cnt: (e, 0)),
        ],
        out_specs=pl.BlockSpec((C, OUT), lambda e, k, r, cnt: (e, 0)),
        scratch_shapes=[pltpu.VMEM((C, OUT), jnp.float32)],
    )
    return pl.pallas_call(
        _ffn_body,
        grid_spec=grid_spec,
        out_shape=jax.ShapeDtypeStruct((EC + C, OUT), jnp.float32),
    )(cnt, disp, w1, b1.reshape(E, KM, 1, BM), w2, b2.reshape(E, 1, OUT),
      scale)


def _combine_body(dst_hbm, sout_hbm, out_hbm, idx_v, rows_v, sem):
    wid = lax.axis_index("s") * 2 + lax.axis_index("c")
    base = wid * ROWS_COMB
    pltpu.sync_copy(dst_hbm.at[pl.ds(base, ROWS_COMB)], idx_v)
    pltpu.async_copy(sout_hbm.at[idx_v], rows_v, sem).wait()
    pltpu.sync_copy(rows_v, out_hbm.at[pl.ds(base, ROWS_COMB)])


def _combine(dst, sout):
    mesh = plsc.VectorSubcoreMesh(core_axis_name="c", subcore_axis_name="s")
    return pl.kernel(
        _combine_body,
        out_type=jax.ShapeDtypeStruct((S, OUT), jnp.float32),
        mesh=mesh,
        compiler_params=pltpu.CompilerParams(needs_layout_passes=False),
        scratch_types=[
            pltpu.VMEM((ROWS_COMB,), jnp.int32),
            pltpu.VMEM((ROWS_COMB, OUT), jnp.float32),
            pltpu.SemaphoreType.DMA,
        ],
    )(dst, sout)


@jax.jit
def kernel(hidden_states, Wg, W1, b1, W2, b2):
    b, t, m = hidden_states.shape
    feat = hidden_states.reshape(S, M)
    wg_pad = jnp.zeros((M, LANES), jnp.float32).at[:, :E].set(Wg)
    dst2, gate2, cnt128, laux = _gating(feat, wg_pad)
    dst = dst2.reshape(S)
    cnt = jnp.concatenate(
        [jnp.minimum(cnt128[0, :E], C), jnp.zeros((1,), jnp.int32)])
    disp, scale = _dispatch(dst, gate2.reshape(S), feat)
    scale_full = jnp.concatenate([scale, jnp.zeros((C,), jnp.float32)])
    sout = _ffn(cnt, disp, W1, b1, W2, b2, scale_full.reshape(EC + C, 1))
    combined = _combine(dst, sout)
    return combined.reshape(b, t, OUT), laux.reshape(())


# speedup vs baseline: 1.1976x; 1.1976x over previous
import functools

import jax
import jax.numpy as jnp
from jax import lax
from jax.experimental import pallas as pl
from jax.experimental.pallas import tpu as pltpu
from jax.experimental.pallas import tpu_sc as plsc

S = 2048
M = 768
E = 8
MID = 3072
OUT = 768
C = 512
EC = E * C
EP = 16
LANES = 128
NW = 32
L = 16
ROWS_DISP = EC // NW
ROWS_COMB = S // NW
BM = 512
KM = MID // BM
SENT_TOK = S
SENT_SLOT = EC


def _gating_body(x_ref, wg_ref, dst_ref, gate_ref, laux_ref):
    x = x_ref[...]
    wg = wg_ref[...]
    logits = jnp.dot(x, wg, preferred_element_type=jnp.float32)
    col = lax.broadcasted_iota(jnp.int32, (S, LANES), 1)
    valid = col < E
    logits = jnp.where(valid, logits, -1e30)
    mx = jnp.max(logits, axis=1, keepdims=True)
    p = jnp.exp(logits - mx)
    probs = p / jnp.sum(p, axis=1, keepdims=True)
    pm = jnp.max(probs, axis=1, keepdims=True)
    is_max = probs == pm
    eidx = jnp.min(jnp.where(is_max, col, LANES), axis=1, keepdims=True)
    onehot = jnp.where((col == eidx) & valid, 1.0, 0.0)
    c = onehot
    k = 1
    while k < S:
        shifted = jnp.concatenate(
            [jnp.zeros((k, LANES), jnp.float32), c[: S - k, :]], axis=0)
        c = c + shifted
        k *= 2
    pos = jnp.sum(c * onehot, axis=1, keepdims=True).astype(jnp.int32) - 1
    kept = pos < C
    dst = jnp.where(kept, eidx * C + pos, SENT_SLOT)
    dst_ref[...] = dst.astype(jnp.int32).reshape(S // LANES, LANES)
    gate = jnp.sum(probs * onehot, axis=1, keepdims=True)
    gate_ref[...] = gate.reshape(S // LANES, LANES)
    counts = jnp.sum(onehot, axis=0, keepdims=True)
    me = jnp.sum(probs, axis=0, keepdims=True) / S
    laux_ref[...] = jnp.sum(me * (counts / S), axis=1, keepdims=True) * E


def _gating(x, wg_pad):
    return pl.pallas_call(
        _gating_body,
        out_shape=(
            jax.ShapeDtypeStruct((S // LANES, LANES), jnp.int32),
            jax.ShapeDtypeStruct((S // LANES, LANES), jnp.float32),
            jax.ShapeDtypeStruct((1, 1), jnp.float32),
        ),
    )(x, wg_pad)


def _dispatch_body(dst_hbm, gate_hbm, feat_hbm, disp_hbm, scale_hbm,
                   dst_v, gate_v, slot_v, scale_v, rows_v, sem):
    wid = lax.axis_index("s") * 2 + lax.axis_index("c")
    base = wid * ROWS_DISP
    pltpu.sync_copy(dst_hbm, dst_v)
    pltpu.sync_copy(gate_hbm, gate_v)
    for j in range(ROWS_DISP // L):
        dflt = (lax.iota(jnp.int32, L) + (base + j * L)) & (S - 1)
        slot_v[pl.ds(j * L, L)] = dflt

    def build(i, _):
        idx = dst_v[pl.ds(i * L, L)] - base
        m = (idx >= 0) & (idx < ROWS_DISP)
        tok = lax.iota(jnp.int32, L) + i * L
        plsc.store_scatter(slot_v, [idx], tok, mask=m)
        return 0
    lax.fori_loop(0, S // L, build, 0)

    for j in range(ROWS_DISP // L):
        sidx = slot_v[pl.ds(j * L, L)]
        scale_v[pl.ds(j * L, L)] = plsc.load_gather(gate_v, [sidx])
    pltpu.sync_copy(scale_v, scale_hbm.at[pl.ds(base, ROWS_DISP)])

    pltpu.async_copy(feat_hbm.at[slot_v], rows_v, sem).wait()
    pltpu.sync_copy(rows_v, disp_hbm.at[pl.ds(base, ROWS_DISP)])


def _dispatch(dst, gate_pad, feat_pad):
    mesh = plsc.VectorSubcoreMesh(core_axis_name="c", subcore_axis_name="s")
    return pl.kernel(
        _dispatch_body,
        out_type=(
            jax.ShapeDtypeStruct((EC, M), jnp.float32),
            jax.ShapeDtypeStruct((EC,), jnp.float32),
        ),
        mesh=mesh,
        compiler_params=pltpu.CompilerParams(needs_layout_passes=False),
        scratch_types=[
            pltpu.VMEM((S,), jnp.int32),
            pltpu.VMEM((S,), jnp.float32),
            pltpu.VMEM((ROWS_DISP,), jnp.int32),
            pltpu.VMEM((ROWS_DISP,), jnp.float32),
            pltpu.VMEM((ROWS_DISP, M), jnp.float32),
            pltpu.SemaphoreType.DMA,
        ],
    )(dst, gate_pad, feat_pad)


def _ffn_body(x_ref, w1_ref, b1_ref, w2_ref, b2_ref, sc_ref, o_ref, acc_ref):
    k = pl.program_id(1)

    @pl.when(k == 0)
    def _():
        acc_ref[...] = jnp.zeros_like(acc_ref)

    h = jnp.dot(x_ref[...].astype(jnp.bfloat16), w1_ref[0].astype(jnp.bfloat16),
                preferred_element_type=jnp.float32)
    h = jnp.maximum(h + b1_ref[0, 0, 0][None, :], 0.0)
    acc_ref[...] += jnp.dot(h.astype(jnp.bfloat16), w2_ref[0].astype(jnp.bfloat16),
                            preferred_element_type=jnp.float32)

    @pl.when(k == KM - 1)
    def _():
        o_ref[...] = (acc_ref[...] + b2_ref[0, 0][None, :]) * sc_ref[...]


def _ffn(disp, w1, b1, w2, b2, scale):
    ce = lambda e: jnp.minimum(e, E - 1)
    return pl.pallas_call(
        _ffn_body,
        grid=(E + 1, KM),
        in_specs=[
            pl.BlockSpec((C, M), lambda e, k: (ce(e), 0)),
            pl.BlockSpec((1, M, BM), lambda e, k: (ce(e), 0, k)),
            pl.BlockSpec((1, 1, 1, BM), lambda e, k: (ce(e), k, 0, 0)),
            pl.BlockSpec((1, BM, OUT), lambda e, k: (ce(e), k, 0)),
            pl.BlockSpec((1, 1, OUT), lambda e, k: (ce(e), 0, 0)),
            pl.BlockSpec((C, 1), lambda e, k: (e, 0)),
        ],
        out_specs=pl.BlockSpec((C, OUT), lambda e, k: (e, 0)),
        out_shape=jax.ShapeDtypeStruct((EC + C, OUT), jnp.float32),
        scratch_shapes=[pltpu.VMEM((C, OUT), jnp.float32)],
    )(disp, w1, b1.reshape(E, KM, 1, BM), w2, b2.reshape(E, 1, OUT), scale)


def _combine_body(dst_hbm, sout_hbm, out_hbm, idx_v, rows_v, sem):
    wid = lax.axis_index("s") * 2 + lax.axis_index("c")
    base = wid * ROWS_COMB
    pltpu.sync_copy(dst_hbm.at[pl.ds(base, ROWS_COMB)], idx_v)
    pltpu.async_copy(sout_hbm.at[idx_v], rows_v, sem).wait()
    pltpu.sync_copy(rows_v, out_hbm.at[pl.ds(base, ROWS_COMB)])


def _combine(dst, sout):
    mesh = plsc.VectorSubcoreMesh(core_axis_name="c", subcore_axis_name="s")
    return pl.kernel(
        _combine_body,
        out_type=jax.ShapeDtypeStruct((S, OUT), jnp.float32),
        mesh=mesh,
        compiler_params=pltpu.CompilerParams(needs_layout_passes=False),
        scratch_types=[
            pltpu.VMEM((ROWS_COMB,), jnp.int32),
            pltpu.VMEM((ROWS_COMB, OUT), jnp.float32),
            pltpu.SemaphoreType.DMA,
        ],
    )(dst, sout)


@jax.jit
def kernel(hidden_states, Wg, W1, b1, W2, b2):
    b, t, m = hidden_states.shape
    feat = hidden_states.reshape(S, M)
    wg_pad = jnp.zeros((M, LANES), jnp.float32).at[:, :E].set(Wg)
    dst2, gate2, laux = _gating(feat, wg_pad)
    dst = dst2.reshape(S)
    disp, scale = _dispatch(dst, gate2.reshape(S), feat)
    scale_full = jnp.concatenate([scale, jnp.zeros((C,), jnp.float32)])
    sout = _ffn(disp, W1, b1, W2, b2, scale_full.reshape(EC + C, 1))
    combined = _combine(dst, sout)
    return combined.reshape(b, t, OUT), laux.reshape(())
